# baseline (speedup 1.0000x reference)
import numpy as np

import jax
import jax.numpy as jnp
from jax import lax
from jax.experimental import pallas as pl
from jax.experimental.pallas import tpu as pltpu
from jax.experimental.pallas import tpu_sc as plsc

N = 10000
E = 320000
D = 128
DP = D // 2
NC = 2
NS = 16
NW = NC * NS
EPT = 10240
C = 64
NCH = EPT // C
NBUF = 2
WCH = 8
NW2 = NCH // WCH
NP = 10240
SLAB = NP // NS
L = 16

_PERM = np.empty((D,), np.int64)
for _j in range(D // 32):
    for _k in range(16):
        _PERM[32 * _j + _k] = 32 * _j + 2 * _k
        _PERM[32 * _j + 16 + _k] = 32 * _j + 2 * _k + 1


def _bcast_lane(v16, i):
    idx = jnp.full((L,), i, jnp.int32)
    return lax.gather(
        v16, idx[:, None],
        lax.GatherDimensionNumbers(offset_dims=(), collapsed_slice_dims=(0,),
                                   start_index_map=(0,)),
        (1,), mode=lax.GatherScatterMode.PROMISE_IN_BOUNDS)


def _sc_body(node_h, src_h, dst_h, attr_h, sums_h, cnts_h,
             acc_sh, srcw, dstw, attrw, rows0_v, rows1_v,
             rowsf_v, cnt_v, sem0, sem1, semw):
    c = lax.axis_index("c")
    s = lax.axis_index("s")
    wid = c * NS + s
    rows = (rows0_v, rows1_v)
    sems = (sem0, sem1)
    zero16 = jnp.zeros((L,), jnp.float32)
    ones16 = jnp.ones((L,), jnp.float32)
    himask = jnp.full((L,), -65536, jnp.int32)

    def _zrow(i, _):
        for j in range(D // L):
            rowsf_v[i, pl.ds(j * L, L)] = zero16
        return 0
    lax.fori_loop(0, C, _zrow, 0)

    def _zcnt(i, _):
        cnt_v[pl.ds(i * L, L)] = zero16
        return 0
    lax.fori_loop(0, NP // L, _zcnt, 0)

    def _zacc(k, _):
        pltpu.sync_copy(rowsf_v, acc_sh.at[pl.ds(s * SLAB + k * C, C)])
        return 0
    lax.fori_loop(0, SLAB // C, _zacc, 0)
    plsc.subcore_barrier()

    def _refill_descs(w, p):
        return (
            pltpu.make_async_copy(src_h.at[wid, w], srcw.at[p], semw),
            pltpu.make_async_copy(dst_h.at[wid, w], dstw.at[p], semw),
            pltpu.make_async_copy(attr_h.at[wid, w], attrw.at[p], semw),
        )

    def _scale_count_scatter(rv, p, k):
        def _grp(g, _):
            a16 = attrw[p, k, pl.ds(g * L, L)]
            d16 = dstw[p, k, pl.ds(g * L, L)]
            plsc.addupdate_scatter(cnt_v, [d16], ones16)
            for i in range(L):
                a = _bcast_lane(a16, i)
                e = g * L + i
                for j in range(DP // L):
                    w16 = rv[e, pl.ds(j * L, L)]
                    lo = lax.bitcast_convert_type(
                        lax.shift_left(w16, 16), jnp.float32)
                    hi = lax.bitcast_convert_type(
                        jnp.bitwise_and(w16, himask), jnp.float32)
                    rowsf_v[e, pl.ds(2 * j * L, L)] = lo * a
                    rowsf_v[e, pl.ds((2 * j + 1) * L, L)] = hi * a
            return 0
        lax.fori_loop(0, C // L, _grp, 0)
        pltpu.sync_copy(rowsf_v, acc_sh.at[dstw.at[p, k]], add=True)

    for d in _refill_descs(0, 0):
        d.start()
        d.wait()
    for b in range(NBUF):
        pltpu.async_copy(node_h.at[srcw.at[0, b]], rows[b], sems[b])

    def _win(w2, _):
        p = jnp.bitwise_and(w2, 1)
        pnext = 1 - p

        @pl.when(w2 + 1 < NW2)
        def _():
            for d in _refill_descs(w2 + 1, pnext):
                d.start()
        for k in range(WCH):
            b = k % NBUF
            rv = rows[b]
            pltpu.make_async_copy(node_h.at[srcw.at[p, k]], rv,
                                  sems[b]).wait()
            _scale_count_scatter(rv, p, k)
            if k == WCH - NBUF - 1:
                @pl.when(w2 + 1 < NW2)
                def _():
                    for d in _refill_descs(w2 + 1, pnext):
                        d.wait()
            cur = w2 * WCH + k
            if k < WCH - NBUF:
                pltpu.async_copy(node_h.at[srcw.at[p, k + NBUF]], rv, sems[b])
            else:
                @pl.when(cur + NBUF < NCH)
                def _():
                    pltpu.async_copy(
                        node_h.at[srcw.at[pnext, k + NBUF - WCH]],
                        rv, sems[b])
        return 0
    lax.fori_loop(0, NW2, _win, 0)
    plsc.subcore_barrier()

    pltpu.sync_copy(acc_sh.at[pl.ds(s * SLAB, SLAB)],
                    sums_h.at[c, pl.ds(s * SLAB, SLAB)])
    pltpu.sync_copy(cnt_v, cnts_h.at[wid])


@jax.jit
def _sc_aggregate(node_pk, src, dst, attr):
    mesh = plsc.VectorSubcoreMesh(core_axis_name="c", subcore_axis_name="s",
                                  num_cores=NC, num_subcores=NS)
    f = pl.kernel(
        _sc_body,
        out_type=[jax.ShapeDtypeStruct((NC, NP, D), jnp.float32),
                  jax.ShapeDtypeStruct((NW, NP), jnp.float32)],
        mesh=mesh,
        compiler_params=pltpu.CompilerParams(needs_layout_passes=False,
                                             use_tc_tiling_on_sc=False),
        scratch_types=[
            pltpu.VMEM_SHARED((NP, D), jnp.float32),
            pltpu.VMEM((2, WCH, C), jnp.int32),
            pltpu.VMEM((2, WCH, C), jnp.int32),
            pltpu.VMEM((2, WCH, C), jnp.float32),
            pltpu.VMEM((C, DP), jnp.int32),
            pltpu.VMEM((C, DP), jnp.int32),
            pltpu.VMEM((C, D), jnp.float32),
            pltpu.VMEM((NP,), jnp.float32),
            pltpu.SemaphoreType.DMA,
            pltpu.SemaphoreType.DMA,
            pltpu.SemaphoreType.DMA,
        ],
    )
    return f(node_pk, src, dst, attr)


def _tc_body(sums_ref, cnts_ref, node_ref, wrel_ref, wroot_ref, w1_ref,
             w2_ref, brel_ref, b1_ref, b2_ref, ln1w_ref, ln1b_ref,
             ln2w_ref, ln2b_ref, out_ref):
    dn = (((1,), (1,)), ((), ()))
    cnt = jnp.clip(jnp.sum(cnts_ref[...], axis=0), 1.0, None)
    agg = (sums_ref[0] + sums_ref[1]) / cnt
    h = (lax.dot_general(agg, wrel_ref[...], dn,
                         preferred_element_type=jnp.float32)
         + brel_ref[...]
         + lax.dot_general(node_ref[...], wroot_ref[...], dn,
                           preferred_element_type=jnp.float32))

    def _ln_relu(t, w, b):
        mu = jnp.mean(t, axis=-1, keepdims=True)
        d = t - mu
        var = jnp.mean(d * d, axis=-1, keepdims=True)
        return jnp.maximum(d * lax.rsqrt(var + 1e-5) * w + b, 0.0)

    t1 = lax.dot_general(h, w1_ref[...], dn,
                         preferred_element_type=jnp.float32) + b1_ref[...]
    y1 = _ln_relu(t1, ln1w_ref[...], ln1b_ref[...])
    t2 = lax.dot_general(y1, w2_ref[...], dn,
                         preferred_element_type=jnp.float32) + b2_ref[...]
    out_ref[...] = _ln_relu(t2, ln2w_ref[...], ln2b_ref[...])


BR = 1024


@jax.jit
def _tc_dense(sums, cnts, node_p, W_rel, W_root, W1, W2,
              b_rel, b1, b2, ln1_w, ln1_b, ln2_w, ln2_b):
    full = pl.BlockSpec((D, D), lambda i: (0, 0))
    vec = pl.BlockSpec((1, D), lambda i: (0, 0))
    return pl.pallas_call(
        _tc_body,
        grid=(NP // BR,),
        in_specs=[
            pl.BlockSpec((NC, BR, D), lambda i: (0, i, 0)),
            pl.BlockSpec((NW, BR, 1), lambda i: (0, i, 0)),
            pl.BlockSpec((BR, D), lambda i: (i, 0)),
            full, full, full, full, vec, vec, vec, vec, vec, vec, vec,
        ],
        out_specs=pl.BlockSpec((BR, D), lambda i: (i, 0)),
        out_shape=jax.ShapeDtypeStruct((NP, D), jnp.float32),
    )(sums, cnts, node_p, W_rel, W_root, W1, W2,
      b_rel.reshape(1, D), b1.reshape(1, D), b2.reshape(1, D),
      ln1_w.reshape(1, D), ln1_b.reshape(1, D),
      ln2_w.reshape(1, D), ln2_b.reshape(1, D))


def kernel(node, edge_index, edge_attr, batch_ptr, W_rel, b_rel, W_root,
           W1, b1, W2, b2, ln1_w, ln1_b, ln2_w, ln2_b):
    pad = NW * EPT - E
    pad_dst = N + (jnp.arange(pad, dtype=jnp.int32) % (NP - N))
    src = jnp.concatenate(
        [edge_index[0], jnp.zeros((pad,), jnp.int32)]).reshape(NW, NW2, WCH, C)
    dst = jnp.concatenate(
        [edge_index[1], pad_dst]).reshape(NW, NW2, WCH, C)
    attr = jnp.concatenate(
        [edge_attr, jnp.zeros((pad,), jnp.float32)]).reshape(NW, NW2, WCH, C)
    node_pk = lax.bitcast_convert_type(
        node.astype(jnp.bfloat16).reshape(N, DP, 2), jnp.int32)
    sums, cnts = _sc_aggregate(node_pk, src, dst, attr)
    node_p = jnp.pad(node, ((0, NP - N), (0, 0)))
    W_rel_p = W_rel[:, _PERM]
    out = _tc_dense(sums, cnts.reshape(NW, NP, 1), node_p,
                    W_rel_p, W_root, W1, W2,
                    b_rel, b1, b2, ln1_w, ln1_b, ln2_w, ln2_b)
    return out[:N]


# speedup vs baseline: 1.4019x; 1.4019x over previous
import numpy as np

import jax
import jax.numpy as jnp
from jax import lax
from jax.experimental import pallas as pl
from jax.experimental.pallas import tpu as pltpu
from jax.experimental.pallas import tpu_sc as plsc

N = 10000
E = 320000
D = 128
DP = D // 2
NC = 2
NS = 16
NW = NC * NS
EPT = 10240
C = 64
NCH = EPT // C
NBUF = 2
WCH = 8
NW2 = NCH // WCH
NP = 10240
SLAB = NP // NS
L = 16

_PERM = np.empty((D,), np.int64)
for _j in range(D // 32):
    for _k in range(16):
        _PERM[32 * _j + _k] = 32 * _j + 2 * _k
        _PERM[32 * _j + 16 + _k] = 32 * _j + 2 * _k + 1


def _bcast_lane(v16, i):
    idx = jnp.full((L,), i, jnp.int32)
    return lax.gather(
        v16, idx[:, None],
        lax.GatherDimensionNumbers(offset_dims=(), collapsed_slice_dims=(0,),
                                   start_index_map=(0,)),
        (1,), mode=lax.GatherScatterMode.PROMISE_IN_BOUNDS)


def _sc_body(node_h, src_h, dst_h, attr_h, sums_h, cnts_h,
             acc_sh, cntall_sh, srcw, dstw, attrw, rows0_v, rows1_v,
             rowsf_v, cnt_v, tmp_v, tacc_v, sem0, sem1, semw):
    c = lax.axis_index("c")
    s = lax.axis_index("s")
    wid = c * NS + s
    rows = (rows0_v, rows1_v)
    sems = (sem0, sem1)
    zero16 = jnp.zeros((L,), jnp.float32)
    ones16 = jnp.ones((L,), jnp.float32)
    himask = jnp.full((L,), -65536, jnp.int32)

    def _zrow(i, _):
        for j in range(D // L):
            rowsf_v[i, pl.ds(j * L, L)] = zero16
        return 0
    lax.fori_loop(0, C, _zrow, 0)

    def _zcnt(i, _):
        cnt_v[pl.ds(i * L, L)] = zero16
        return 0
    lax.fori_loop(0, NP // L, _zcnt, 0)

    def _zacc(k, _):
        pltpu.sync_copy(rowsf_v, acc_sh.at[pl.ds(s * SLAB + k * C, C)])
        return 0
    lax.fori_loop(0, SLAB // C, _zacc, 0)
    plsc.subcore_barrier()

    def _refill_descs(w, p):
        return (
            pltpu.make_async_copy(src_h.at[wid, w], srcw.at[p], semw),
            pltpu.make_async_copy(dst_h.at[wid, w], dstw.at[p], semw),
            pltpu.make_async_copy(attr_h.at[wid, w], attrw.at[p], semw),
        )

    def _scale_count_scatter(rv, p, k):
        def _grp(g, _):
            a16 = attrw[p, k, pl.ds(g * L, L)]
            d16 = dstw[p, k, pl.ds(g * L, L)]
            plsc.addupdate_scatter(cnt_v, [d16], ones16)
            for i in range(L):
                a = _bcast_lane(a16, i)
                e = g * L + i
                for j in range(DP // L):
                    w16 = rv[e, pl.ds(j * L, L)]
                    lo = lax.bitcast_convert_type(
                        lax.shift_left(w16, 16), jnp.float32)
                    hi = lax.bitcast_convert_type(
                        jnp.bitwise_and(w16, himask), jnp.float32)
                    rowsf_v[e, pl.ds(2 * j * L, L)] = lo * a
                    rowsf_v[e, pl.ds((2 * j + 1) * L, L)] = hi * a
            return 0
        lax.fori_loop(0, C // L, _grp, 0)
        pltpu.sync_copy(rowsf_v, acc_sh.at[dstw.at[p, k]], add=True)

    for d in _refill_descs(0, 0):
        d.start()
        d.wait()
    for b in range(NBUF):
        pltpu.async_copy(node_h.at[srcw.at[0, b]], rows[b], sems[b])

    def _win(w2, _):
        p = jnp.bitwise_and(w2, 1)
        pnext = 1 - p

        @pl.when(w2 + 1 < NW2)
        def _():
            for d in _refill_descs(w2 + 1, pnext):
                d.start()
        for k in range(WCH):
            b = k % NBUF
            rv = rows[b]
            pltpu.make_async_copy(node_h.at[srcw.at[p, k]], rv,
                                  sems[b]).wait()
            _scale_count_scatter(rv, p, k)
            if k == WCH - NBUF - 1:
                @pl.when(w2 + 1 < NW2)
                def _():
                    for d in _refill_descs(w2 + 1, pnext):
                        d.wait()
            cur = w2 * WCH + k
            if k < WCH - NBUF:
                pltpu.async_copy(node_h.at[srcw.at[p, k + NBUF]], rv, sems[b])
            else:
                @pl.when(cur + NBUF < NCH)
                def _():
                    pltpu.async_copy(
                        node_h.at[srcw.at[pnext, k + NBUF - WCH]],
                        rv, sems[b])
        return 0
    lax.fori_loop(0, NW2, _win, 0)
    plsc.subcore_barrier()

    pltpu.sync_copy(acc_sh.at[pl.ds(s * SLAB, SLAB)],
                    sums_h.at[c, pl.ds(s * SLAB, SLAB)])

    pltpu.sync_copy(cnt_v, cntall_sh.at[s])
    plsc.subcore_barrier()

    def _ztacc(i, _):
        tacc_v[pl.ds(i * L, L)] = zero16
        return 0
    lax.fori_loop(0, SLAB // L, _ztacc, 0)

    def _red(t, _):
        pltpu.sync_copy(cntall_sh.at[t, pl.ds(s * SLAB, SLAB)], tmp_v)

        def _add(k, _):
            sl = pl.ds(k * L, L)
            tacc_v[sl] = tacc_v[sl] + tmp_v[sl]
            return 0
        lax.fori_loop(0, SLAB // L, _add, 0)
        return 0
    lax.fori_loop(0, NS, _red, 0)
    pltpu.sync_copy(tacc_v, cnts_h.at[c, pl.ds(s * SLAB, SLAB)])


@jax.jit
def _sc_aggregate(node_pk, src, dst, attr):
    mesh = plsc.VectorSubcoreMesh(core_axis_name="c", subcore_axis_name="s",
                                  num_cores=NC, num_subcores=NS)
    f = pl.kernel(
        _sc_body,
        out_type=[jax.ShapeDtypeStruct((NC, NP, D), jnp.float32),
                  jax.ShapeDtypeStruct((NC, NP), jnp.float32)],
        mesh=mesh,
        compiler_params=pltpu.CompilerParams(needs_layout_passes=False,
                                             use_tc_tiling_on_sc=False),
        scratch_types=[
            pltpu.VMEM_SHARED((NP, D), jnp.float32),
            pltpu.VMEM_SHARED((NS, NP), jnp.float32),
            pltpu.VMEM((2, WCH, C), jnp.int32),
            pltpu.VMEM((2, WCH, C), jnp.int32),
            pltpu.VMEM((2, WCH, C), jnp.float32),
            pltpu.VMEM((C, DP), jnp.int32),
            pltpu.VMEM((C, DP), jnp.int32),
            pltpu.VMEM((C, D), jnp.float32),
            pltpu.VMEM((NP,), jnp.float32),
            pltpu.VMEM((SLAB,), jnp.float32),
            pltpu.VMEM((SLAB,), jnp.float32),
            pltpu.SemaphoreType.DMA,
            pltpu.SemaphoreType.DMA,
            pltpu.SemaphoreType.DMA,
        ],
    )
    return f(node_pk, src, dst, attr)


def _tc_body(sums_ref, cnts_ref, node_ref, wrel_ref, wroot_ref, w1_ref,
             w2_ref, brel_ref, b1_ref, b2_ref, ln1w_ref, ln1b_ref,
             ln2w_ref, ln2b_ref, out_ref):
    dn = (((1,), (1,)), ((), ()))
    cnt = jnp.clip(cnts_ref[0] + cnts_ref[1], 1.0, None)
    agg = (sums_ref[0] + sums_ref[1]) / cnt
    h = (lax.dot_general(agg, wrel_ref[...], dn,
                         preferred_element_type=jnp.float32)
         + brel_ref[...]
         + lax.dot_general(node_ref[...], wroot_ref[...], dn,
                           preferred_element_type=jnp.float32))

    def _ln_relu(t, w, b):
        mu = jnp.mean(t, axis=-1, keepdims=True)
        d = t - mu
        var = jnp.mean(d * d, axis=-1, keepdims=True)
        return jnp.maximum(d * lax.rsqrt(var + 1e-5) * w + b, 0.0)

    t1 = lax.dot_general(h, w1_ref[...], dn,
                         preferred_element_type=jnp.float32) + b1_ref[...]
    y1 = _ln_relu(t1, ln1w_ref[...], ln1b_ref[...])
    t2 = lax.dot_general(y1, w2_ref[...], dn,
                         preferred_element_type=jnp.float32) + b2_ref[...]
    out_ref[...] = _ln_relu(t2, ln2w_ref[...], ln2b_ref[...])


BR = 1024


@jax.jit
def _tc_dense(sums, cnts, node_p, W_rel, W_root, W1, W2,
              b_rel, b1, b2, ln1_w, ln1_b, ln2_w, ln2_b):
    full = pl.BlockSpec((D, D), lambda i: (0, 0))
    vec = pl.BlockSpec((1, D), lambda i: (0, 0))
    return pl.pallas_call(
        _tc_body,
        grid=(NP // BR,),
        in_specs=[
            pl.BlockSpec((NC, BR, D), lambda i: (0, i, 0)),
            pl.BlockSpec((NC, BR, 1), lambda i: (0, i, 0)),
            pl.BlockSpec((BR, D), lambda i: (i, 0)),
            full, full, full, full, vec, vec, vec, vec, vec, vec, vec,
        ],
        out_specs=pl.BlockSpec((BR, D), lambda i: (i, 0)),
        out_shape=jax.ShapeDtypeStruct((NP, D), jnp.float32),
    )(sums, cnts, node_p, W_rel, W_root, W1, W2,
      b_rel.reshape(1, D), b1.reshape(1, D), b2.reshape(1, D),
      ln1_w.reshape(1, D), ln1_b.reshape(1, D),
      ln2_w.reshape(1, D), ln2_b.reshape(1, D))


def kernel(node, edge_index, edge_attr, batch_ptr, W_rel, b_rel, W_root,
           W1, b1, W2, b2, ln1_w, ln1_b, ln2_w, ln2_b):
    pad = NW * EPT - E
    pad_dst = N + (jnp.arange(pad, dtype=jnp.int32) % (NP - N))
    src = jnp.concatenate(
        [edge_index[0], jnp.zeros((pad,), jnp.int32)]).reshape(NW, NW2, WCH, C)
    dst = jnp.concatenate(
        [edge_index[1], pad_dst]).reshape(NW, NW2, WCH, C)
    attr = jnp.concatenate(
        [edge_attr, jnp.zeros((pad,), jnp.float32)]).reshape(NW, NW2, WCH, C)
    node_pk = lax.bitcast_convert_type(
        node.astype(jnp.bfloat16).reshape(N, DP, 2), jnp.int32)
    sums, cnts = _sc_aggregate(node_pk, src, dst, attr)
    node_p = jnp.pad(node, ((0, NP - N), (0, 0)))
    W_rel_p = W_rel[:, _PERM]
    out = _tc_dense(sums, cnts.reshape(NC, NP, 1), node_p,
                    W_rel_p, W_root, W1, W2,
                    b_rel, b1, b2, ln1_w, ln1_b, ln2_w, ln2_b)
    return out[:N]
